# Initial kernel scaffold; baseline (speedup 1.0000x reference)
import jax
import jax.numpy as jnp
from jax.experimental import pallas as pl


def kernel(h, edge_index, W, a):
    raise NotImplementedError("write your pallas kernel here")


# speedup vs baseline: 3.5561x; 3.5561x over previous
import functools

import jax
import jax.numpy as jnp
from jax import lax
from jax.experimental import pallas as pl
from jax.experimental.pallas import tpu as pltpu
from jax.experimental.pallas import tpu_sc as plsc

N = 10000
E = 160000
IN_F = 256
OUT_F = 512

NC = 2
NS = 16
NW = NC * NS
G = 128
NPAD = 10240
EPAD = 163840
NG = EPAD // (NW * G)
NCHUNK = 4
CW = OUT_F // NCHUNK
RPT = NPAD // NS
ZR = 16
RB = 512
GRID = NPAD // RB


def _mm_body(h_ref, w_ref, a2_ref, z0_ref, z1_ref, z2_ref, z3_ref, s_ref):
    z = jnp.dot(h_ref[...], w_ref[...], preferred_element_type=jnp.float32)
    zrefs = (z0_ref, z1_ref, z2_ref, z3_ref)
    for c in range(NCHUNK):
        zrefs[c][...] = z[:, c * CW:(c + 1) * CW]
    s_ref[...] = lax.dot_general(a2_ref[...], z, (((0,), (1,)), ((), ())),
                                 preferred_element_type=jnp.float32)


def _tc_matmul(h, W, A2):
    return pl.pallas_call(
        _mm_body,
        grid=(GRID,),
        in_specs=[
            pl.BlockSpec((RB, IN_F), lambda i: (i, 0)),
            pl.BlockSpec((IN_F, OUT_F), lambda i: (0, 0)),
            pl.BlockSpec((OUT_F, 2), lambda i: (0, 0)),
        ],
        out_specs=[pl.BlockSpec((RB, CW), lambda i: (i, 0))] * NCHUNK
        + [pl.BlockSpec((2, RB), lambda i: (0, i))],
        out_shape=[jax.ShapeDtypeStruct((NPAD, CW), jnp.float32)] * NCHUNK
        + [jax.ShapeDtypeStruct((2, NPAD), jnp.float32)],
    )(h, W, A2)


def _sca_body(s1_hbm, s2_hbm, src_hbm, dst_hbm,
              w_hbm,
              src_v, dst_v, w_v, s1_v, s2_v):
    cid = lax.axis_index("c")
    sid = lax.axis_index("s")
    wid = cid * NS + sid

    pltpu.sync_copy(src_hbm.at[wid], src_v)
    pltpu.sync_copy(dst_hbm.at[wid], dst_v)
    pltpu.sync_copy(s1_hbm, s1_v)
    pltpu.sync_copy(s2_hbm, s2_v)

    def _w_body(g, carry):
        for k in range(G // 16):
            sl = pl.ds(k * 16, 16)
            idx_s = src_v[g, sl]
            idx_d = dst_v[g, sl]
            e = plsc.load_gather(s1_v, [idx_s]) + plsc.load_gather(s2_v, [idx_d])
            e = jnp.where(e > 0, e, e * jnp.float32(0.01))
            w_v[g, sl] = jnp.exp(e)
        return carry
    lax.fori_loop(0, NG, _w_body, 0)

    pltpu.sync_copy(w_v, w_hbm.at[wid])


_sc_weights = functools.partial(
    pl.kernel,
    out_type=jax.ShapeDtypeStruct((NW, NG, G), jnp.float32),
    mesh=plsc.VectorSubcoreMesh(core_axis_name="c", subcore_axis_name="s"),
    compiler_params=pltpu.CompilerParams(needs_layout_passes=False),
    scratch_types=[
        pltpu.VMEM((NG, G), jnp.int32),
        pltpu.VMEM((NG, G), jnp.int32),
        pltpu.VMEM((NG, G), jnp.float32),
        pltpu.VMEM((NPAD,), jnp.float32),
        pltpu.VMEM((NPAD,), jnp.float32),
    ],
)(_sca_body)


def _scb_body(z0, z1, z2, z3, w_hbm, src_hbm, dst_hbm,
              num_hbm, den_hbm,
              src_v, dst_v, w_v, rows_v, zrows_v, acc):
    cid = lax.axis_index("c")
    sid = lax.axis_index("s")
    wid = cid * NS + sid

    zs = (z0, z1, z2, z3)

    pltpu.sync_copy(src_hbm.at[wid], src_v)
    pltpu.sync_copy(dst_hbm.at[wid], dst_v)
    pltpu.sync_copy(w_hbm.at[wid], w_v)

    zero16 = jnp.zeros((16,), jnp.float32)
    ones16 = jnp.ones((16,), jnp.float32)

    def _zrow_body(j, carry):
        for k in range(CW // 16):
            zrows_v[j, pl.ds(k * 16, 16)] = zero16
        return carry
    lax.fori_loop(0, ZR, _zrow_body, 0)

    row0 = sid * RPT
    for c in range(NCHUNK + 1):
        for b in range(RPT // ZR):
            pltpu.sync_copy(zrows_v, acc.at[pl.ds(row0 + b * ZR, ZR)])
        plsc.subcore_barrier()

        if c < NCHUNK:
            def _g_body(g, carry):
                pltpu.sync_copy(zs[c].at[src_v.at[g]], rows_v)

                def _r_body(r16, rcarry):
                    wv = w_v[g, pl.ds(r16 * 16, 16)]
                    for j in range(16):
                        w = wv[j]
                        r = r16 * 16 + j
                        for k in range(CW // 16):
                            sl = pl.ds(k * 16, 16)
                            rows_v[r, sl] = rows_v[r, sl] * w
                    return rcarry
                lax.fori_loop(0, G // 16, _r_body, 0)
                pltpu.sync_copy(rows_v, acc.at[dst_v.at[g]], add=True)
                return carry
        else:
            def _g_body(g, carry):
                def _r_body(r16, rcarry):
                    wv = w_v[g, pl.ds(r16 * 16, 16)]
                    for j in range(16):
                        bw = wv[j] * ones16
                        r = r16 * 16 + j
                        for k in range(CW // 16):
                            rows_v[r, pl.ds(k * 16, 16)] = bw
                    return rcarry
                lax.fori_loop(0, G // 16, _r_body, 0)
                pltpu.sync_copy(rows_v, acc.at[dst_v.at[g]], add=True)
                return carry
        lax.fori_loop(0, NG, _g_body, 0)

        plsc.subcore_barrier()
        if c < NCHUNK:
            pltpu.sync_copy(acc.at[pl.ds(row0, RPT)],
                            num_hbm.at[cid, c, pl.ds(row0, RPT)])
        else:
            pltpu.sync_copy(acc.at[pl.ds(row0, RPT)],
                            den_hbm.at[cid, pl.ds(row0, RPT)])
        if c < NCHUNK:
            plsc.subcore_barrier()


_sc_aggregate = functools.partial(
    pl.kernel,
    out_type=[jax.ShapeDtypeStruct((NC, NCHUNK, NPAD, CW), jnp.float32),
              jax.ShapeDtypeStruct((NC, NPAD, CW), jnp.float32)],
    mesh=plsc.VectorSubcoreMesh(core_axis_name="c", subcore_axis_name="s"),
    compiler_params=pltpu.CompilerParams(needs_layout_passes=False),
    scratch_types=[
        pltpu.VMEM((NG, G), jnp.int32),
        pltpu.VMEM((NG, G), jnp.int32),
        pltpu.VMEM((NG, G), jnp.float32),
        pltpu.VMEM((G, CW), jnp.float32),
        pltpu.VMEM((ZR, CW), jnp.float32),
        pltpu.VMEM_SHARED((NPAD, CW), jnp.float32),
    ],
)(_scb_body)


def _comb_body(num_ref, den_ref, out_ref):
    d = den_ref[0, :, 0:1] + den_ref[1, :, 0:1]
    good = d > 0
    dsafe = jnp.where(good, d, jnp.float32(1.0))
    for c in range(NCHUNK):
        n = num_ref[0, c] + num_ref[1, c]
        out_ref[:, c * CW:(c + 1) * CW] = jnp.where(good, n / dsafe,
                                                    jnp.float32(0.0))


def _tc_combine(num, den):
    return pl.pallas_call(
        _comb_body,
        grid=(GRID,),
        in_specs=[
            pl.BlockSpec((NC, NCHUNK, RB, CW), lambda i: (0, 0, i, 0)),
            pl.BlockSpec((NC, RB, CW), lambda i: (0, i, 0)),
        ],
        out_specs=pl.BlockSpec((RB, OUT_F), lambda i: (i, 0)),
        out_shape=jax.ShapeDtypeStruct((N, OUT_F), jnp.float32),
    )(num, den)


def kernel(h, edge_index, W, a):
    src = edge_index[0]
    dst = edge_index[1]
    pad = jnp.full((EPAD - E,), NPAD - 1, jnp.int32)
    src3 = jnp.concatenate([src, pad]).reshape(NW, NG, G)
    dst3 = jnp.concatenate([dst, pad]).reshape(NW, NG, G)
    A2 = jnp.concatenate([a[:OUT_F], a[OUT_F:]], axis=1)

    z0, z1, z2, z3, s = _tc_matmul(h, W, A2)
    w3 = _sc_weights(s[0], s[1], src3, dst3)
    num, den = _sc_aggregate(z0, z1, z2, z3, w3, src3, dst3)
    return _tc_combine(num, den)


# speedup vs baseline: 4.0854x; 1.1488x over previous
import functools

import jax
import jax.numpy as jnp
from jax import lax
from jax.experimental import pallas as pl
from jax.experimental.pallas import tpu as pltpu
from jax.experimental.pallas import tpu_sc as plsc

N = 10000
E = 160000
IN_F = 256
OUT_F = 512

NC = 2
NS = 16
NW = NC * NS
G = 128
NPAD = 10240
EPAD = 163840
NG = EPAD // (NW * G)
NCHUNK = 4
CW = OUT_F // NCHUNK
RPT = NPAD // NS
ZR = 16
RB = 512
GRID = NPAD // RB


def _mm_body(h_ref, w_ref, a2_ref, z0_ref, z1_ref, z2_ref, z3_ref, s_ref):
    z = jnp.dot(h_ref[...], w_ref[...], preferred_element_type=jnp.float32)
    zrefs = (z0_ref, z1_ref, z2_ref, z3_ref)
    for c in range(NCHUNK):
        zrefs[c][...] = z[:, c * CW:(c + 1) * CW]
    s_ref[...] = lax.dot_general(a2_ref[...], z, (((0,), (1,)), ((), ())),
                                 preferred_element_type=jnp.float32)


def _tc_matmul(h, W, A2):
    return pl.pallas_call(
        _mm_body,
        grid=(GRID,),
        in_specs=[
            pl.BlockSpec((RB, IN_F), lambda i: (i, 0)),
            pl.BlockSpec((IN_F, OUT_F), lambda i: (0, 0)),
            pl.BlockSpec((OUT_F, 2), lambda i: (0, 0)),
        ],
        out_specs=[pl.BlockSpec((RB, CW), lambda i: (i, 0))] * NCHUNK
        + [pl.BlockSpec((2, RB), lambda i: (0, i))],
        out_shape=[jax.ShapeDtypeStruct((NPAD, CW), jnp.float32)] * NCHUNK
        + [jax.ShapeDtypeStruct((2, NPAD), jnp.float32)],
    )(h, W, A2)


def _sca_body(s1_hbm, s2_hbm, src_hbm, dst_hbm,
              w_hbm,
              src_v, dst_v, w_v, s1_v, s2_v):
    cid = lax.axis_index("c")
    sid = lax.axis_index("s")
    wid = cid * NS + sid

    pltpu.sync_copy(src_hbm.at[wid], src_v)
    pltpu.sync_copy(dst_hbm.at[wid], dst_v)
    pltpu.sync_copy(s1_hbm, s1_v)
    pltpu.sync_copy(s2_hbm, s2_v)

    def _w_body(g, carry):
        for k in range(G // 16):
            sl = pl.ds(k * 16, 16)
            idx_s = src_v[g, sl]
            idx_d = dst_v[g, sl]
            e = plsc.load_gather(s1_v, [idx_s]) + plsc.load_gather(s2_v, [idx_d])
            e = jnp.where(e > 0, e, e * jnp.float32(0.01))
            w_v[g, sl] = jnp.exp(e)
        return carry
    lax.fori_loop(0, NG, _w_body, 0)

    pltpu.sync_copy(w_v, w_hbm.at[wid])


_sc_weights = functools.partial(
    pl.kernel,
    out_type=jax.ShapeDtypeStruct((NW, NG, G), jnp.float32),
    mesh=plsc.VectorSubcoreMesh(core_axis_name="c", subcore_axis_name="s"),
    compiler_params=pltpu.CompilerParams(needs_layout_passes=False),
    scratch_types=[
        pltpu.VMEM((NG, G), jnp.int32),
        pltpu.VMEM((NG, G), jnp.int32),
        pltpu.VMEM((NG, G), jnp.float32),
        pltpu.VMEM((NPAD,), jnp.float32),
        pltpu.VMEM((NPAD,), jnp.float32),
    ],
)(_sca_body)


def _scb_body(z0, z1, z2, z3, w_hbm, src_hbm, dst_hbm, zer_hbm,
              num_hbm, den_hbm,
              src_v, dst_v, w_v, r0_v, r1_v, acc,
              gs0, gs1, ss0, ss1):
    cid = lax.axis_index("c")
    sid = lax.axis_index("s")
    wid = cid * NS + sid

    zs = (z0, z1, z2, z3)

    pltpu.sync_copy(src_hbm.at[wid], src_v)
    pltpu.sync_copy(dst_hbm.at[wid], dst_v)
    pltpu.sync_copy(w_hbm.at[wid], w_v)

    ones16 = jnp.ones((16,), jnp.float32)
    row0 = sid * RPT

    def _scale(g, buf):
        def _r_body(r16, rcarry):
            wv = w_v[g, pl.ds(r16 * 16, 16)]
            for j in range(16):
                w = wv[j]
                r = r16 * 16 + j
                for k in range(CW // 16):
                    sl = pl.ds(k * 16, 16)
                    buf[r, sl] = buf[r, sl] * w
            return rcarry
        lax.fori_loop(0, G // 16, _r_body, 0)

    def _fill(g, buf):
        def _r_body(r16, rcarry):
            wv = w_v[g, pl.ds(r16 * 16, 16)]
            for j in range(16):
                bw = wv[j] * ones16
                r = r16 * 16 + j
                for k in range(CW // 16):
                    buf[r, pl.ds(k * 16, 16)] = bw
            return rcarry
        lax.fori_loop(0, G // 16, _r_body, 0)

    def _gather(c, g, buf, sem):
        pltpu.async_copy(zs[c].at[src_v.at[g]], buf, sem)

    def _wait_gather(c, g, buf, sem):
        pltpu.make_async_copy(zs[c].at[src_v.at[g]], buf, sem).wait()

    def _scatter(g, buf, sem):
        pltpu.async_copy(buf, acc.at[dst_v.at[g]], sem, add=True)

    def _wait_scatter(g, buf, sem):
        pltpu.make_async_copy(buf, acc.at[dst_v.at[g]], sem).wait()

    for c in range(NCHUNK + 1):
        pltpu.sync_copy(zer_hbm.at[pl.ds(row0, RPT)],
                        acc.at[pl.ds(row0, RPT)])
        plsc.subcore_barrier()

        if c < NCHUNK:
            _gather(c, 0, r0_v, gs0)

            def _t_body(t, carry):
                g0 = 2 * t
                g1 = g0 + 1
                _wait_gather(c, g0, r0_v, gs0)

                @pl.when(t > 0)
                def _():
                    _wait_scatter(g1 - 2, r1_v, ss1)
                _gather(c, g1, r1_v, gs1)
                _scale(g0, r0_v)
                _scatter(g0, r0_v, ss0)
                _wait_gather(c, g1, r1_v, gs1)

                @pl.when(t < NG // 2 - 1)
                def _():
                    _wait_scatter(g0, r0_v, ss0)
                    _gather(c, g0 + 2, r0_v, gs0)
                _scale(g1, r1_v)
                _scatter(g1, r1_v, ss1)
                return carry
            lax.fori_loop(0, NG // 2, _t_body, 0)
            _wait_scatter(NG - 2, r0_v, ss0)
            _wait_scatter(NG - 1, r1_v, ss1)
        else:
            def _t_body(t, carry):
                g0 = 2 * t
                g1 = g0 + 1

                @pl.when(t > 0)
                def _():
                    _wait_scatter(g0 - 2, r0_v, ss0)
                _fill(g0, r0_v)
                _scatter(g0, r0_v, ss0)

                @pl.when(t > 0)
                def _():
                    _wait_scatter(g1 - 2, r1_v, ss1)
                _fill(g1, r1_v)
                _scatter(g1, r1_v, ss1)
                return carry
            lax.fori_loop(0, NG // 2, _t_body, 0)
            _wait_scatter(NG - 2, r0_v, ss0)
            _wait_scatter(NG - 1, r1_v, ss1)

        plsc.subcore_barrier()
        if c < NCHUNK:
            pltpu.sync_copy(acc.at[pl.ds(row0, RPT)],
                            num_hbm.at[cid, c, pl.ds(row0, RPT)])
        else:
            pltpu.sync_copy(acc.at[pl.ds(row0, RPT)],
                            den_hbm.at[cid, pl.ds(row0, RPT)])
        if c < NCHUNK:
            plsc.subcore_barrier()


_sc_aggregate = functools.partial(
    pl.kernel,
    out_type=[jax.ShapeDtypeStruct((NC, NCHUNK, NPAD, CW), jnp.float32),
              jax.ShapeDtypeStruct((NC, NPAD, CW), jnp.float32)],
    mesh=plsc.VectorSubcoreMesh(core_axis_name="c", subcore_axis_name="s"),
    compiler_params=pltpu.CompilerParams(needs_layout_passes=False),
    scratch_types=[
        pltpu.VMEM((NG, G), jnp.int32),
        pltpu.VMEM((NG, G), jnp.int32),
        pltpu.VMEM((NG, G), jnp.float32),
        pltpu.VMEM((G, CW), jnp.float32),
        pltpu.VMEM((G, CW), jnp.float32),
        pltpu.VMEM_SHARED((NPAD, CW), jnp.float32),
        pltpu.SemaphoreType.DMA,
        pltpu.SemaphoreType.DMA,
        pltpu.SemaphoreType.DMA,
        pltpu.SemaphoreType.DMA,
    ],
)(_scb_body)


def _comb_body(num_ref, den_ref, out_ref):
    d = den_ref[0, :, 0:1] + den_ref[1, :, 0:1]
    good = d > 0
    dsafe = jnp.where(good, d, jnp.float32(1.0))
    for c in range(NCHUNK):
        n = num_ref[0, c] + num_ref[1, c]
        out_ref[:, c * CW:(c + 1) * CW] = jnp.where(good, n / dsafe,
                                                    jnp.float32(0.0))


def _tc_combine(num, den):
    return pl.pallas_call(
        _comb_body,
        grid=(GRID,),
        in_specs=[
            pl.BlockSpec((NC, NCHUNK, RB, CW), lambda i: (0, 0, i, 0)),
            pl.BlockSpec((NC, RB, CW), lambda i: (0, i, 0)),
        ],
        out_specs=pl.BlockSpec((RB, OUT_F), lambda i: (i, 0)),
        out_shape=jax.ShapeDtypeStruct((N, OUT_F), jnp.float32),
    )(num, den)


def kernel(h, edge_index, W, a):
    src = edge_index[0]
    dst = edge_index[1]
    pad = jnp.full((EPAD - E,), NPAD - 1, jnp.int32)
    src3 = jnp.concatenate([src, pad]).reshape(NW, NG, G)
    dst3 = jnp.concatenate([dst, pad]).reshape(NW, NG, G)
    A2 = jnp.concatenate([a[:OUT_F], a[OUT_F:]], axis=1)

    z0, z1, z2, z3, s = _tc_matmul(h, W, A2)
    w3 = _sc_weights(s[0], s[1], src3, dst3)
    zer = jnp.zeros((NPAD, CW), jnp.float32)
    num, den = _sc_aggregate(z0, z1, z2, z3, w3, src3, dst3, zer)
    return _tc_combine(num, den)


# speedup vs baseline: 4.2298x; 1.0354x over previous
import functools

import jax
import jax.numpy as jnp
from jax import lax
from jax.experimental import pallas as pl
from jax.experimental.pallas import tpu as pltpu
from jax.experimental.pallas import tpu_sc as plsc

N = 10000
E = 160000
IN_F = 256
OUT_F = 512

NC = 2
NS = 16
NW = NC * NS
NPAD = 10240
EPAD = 163840
NCHUNK = 4
CW = OUT_F // NCHUNK
RPT = NPAD // NS
RB = 512
GRID = NPAD // RB

GA = 128
NGA = EPAD // (NW * GA)
EW = NGA * GA

G = 64
NGROUPS = EPAD // G
GPP = NGROUPS // NS
K0 = 118
K1 = GPP - K0
KMAX = max(K0, K1)


def _mm_body(h_ref, w_ref, a2_ref, z0_ref, z1_ref, z2_ref, z3_ref, s_ref):
    z = jnp.dot(h_ref[...], w_ref[...], preferred_element_type=jnp.float32)
    zrefs = (z0_ref, z1_ref, z2_ref, z3_ref)
    for c in range(NCHUNK):
        zrefs[c][...] = z[:, c * CW:(c + 1) * CW]
    s_ref[...] = lax.dot_general(a2_ref[...], z, (((0,), (1,)), ((), ())),
                                 preferred_element_type=jnp.float32)


def _tc_matmul(h, W, A2):
    return pl.pallas_call(
        _mm_body,
        grid=(GRID,),
        in_specs=[
            pl.BlockSpec((RB, IN_F), lambda i: (i, 0)),
            pl.BlockSpec((IN_F, OUT_F), lambda i: (0, 0)),
            pl.BlockSpec((OUT_F, 2), lambda i: (0, 0)),
        ],
        out_specs=[pl.BlockSpec((RB, CW), lambda i: (i, 0))] * NCHUNK
        + [pl.BlockSpec((2, RB), lambda i: (0, i))],
        out_shape=[jax.ShapeDtypeStruct((NPAD, CW), jnp.float32)] * NCHUNK
        + [jax.ShapeDtypeStruct((2, NPAD), jnp.float32)],
    )(h, W, A2)


def _sca_body(s1_hbm, s2_hbm, src_hbm, dst_hbm,
              w_hbm,
              src_v, dst_v, w_v, s1_v, s2_v):
    cid = lax.axis_index("c")
    sid = lax.axis_index("s")
    wid = cid * NS + sid

    pltpu.sync_copy(src_hbm.at[wid], src_v)
    pltpu.sync_copy(dst_hbm.at[wid], dst_v)
    pltpu.sync_copy(s1_hbm, s1_v)
    pltpu.sync_copy(s2_hbm, s2_v)

    def _w_body(g, carry):
        for k in range(GA // 16):
            sl = pl.ds(k * 16, 16)
            idx_s = src_v[g, sl]
            idx_d = dst_v[g, sl]
            e = plsc.load_gather(s1_v, [idx_s]) + plsc.load_gather(s2_v, [idx_d])
            e = jnp.where(e > 0, e, e * jnp.float32(0.01))
            w_v[pl.ds(g * GA + k * 16, 16)] = jnp.exp(e)
        return carry
    lax.fori_loop(0, NGA, _w_body, 0)

    pltpu.sync_copy(w_v, w_hbm.at[pl.ds(wid * EW, EW)])


_sc_weights = functools.partial(
    pl.kernel,
    out_type=jax.ShapeDtypeStruct((EPAD,), jnp.float32),
    mesh=plsc.VectorSubcoreMesh(core_axis_name="c", subcore_axis_name="s"),
    compiler_params=pltpu.CompilerParams(needs_layout_passes=False),
    scratch_types=[
        pltpu.VMEM((NGA, GA), jnp.int32),
        pltpu.VMEM((NGA, GA), jnp.int32),
        pltpu.VMEM((EW,), jnp.float32),
        pltpu.VMEM((NPAD,), jnp.float32),
        pltpu.VMEM((NPAD,), jnp.float32),
    ],
)(_sca_body)


def _scb_body(z0, z1, z2, z3, w_hbm, src_hbm, dst_hbm, zer_hbm,
              num_hbm, den_hbm,
              src_v, w_sl, g0_v, g1_v, s0_v, s1_v, db_v, acc,
              gs0, gs1, ss0, ss1, sd0, sd1, sd2, sd3):
    cid = lax.axis_index("c")
    sid = lax.axis_index("s")

    zs = (z0, z1, z2, z3)
    sds = (sd0, sd1, sd2, sd3)

    k_here = jnp.where(cid == 0, K0, K1)
    gstart = jnp.where(cid == 0, sid * K0, NS * K0 + sid * K1)

    @pl.when(cid == 0)
    def _():
        pltpu.sync_copy(src_hbm.at[pl.ds(gstart * G, K0 * G)],
                        src_v.at[pl.ds(0, K0 * G)])
        pltpu.sync_copy(w_hbm.at[pl.ds(gstart * G, K0 * G)],
                        w_sl.at[pl.ds(0, K0 * G)])

    @pl.when(cid == 1)
    def _():
        pltpu.sync_copy(src_hbm.at[pl.ds(gstart * G, K1 * G)],
                        src_v.at[pl.ds(0, K1 * G)])
        pltpu.sync_copy(w_hbm.at[pl.ds(gstart * G, K1 * G)],
                        w_sl.at[pl.ds(0, K1 * G)])

    ones16 = jnp.ones((16,), jnp.float32)
    row0 = sid * RPT

    def _scale_into(l, gbuf, sbuf):
        def _r_body(r, rcarry):
            wv = plsc.load_gather(w_sl, [jnp.full((16,), l * G + r, jnp.int32)])
            for k in range(CW // 16):
                sl = pl.ds(k * 16, 16)
                sbuf[r, sl] = gbuf[r, sl] * wv
            return rcarry
        lax.fori_loop(0, G, _r_body, 0)

    def _fill(l, sbuf):
        def _r_body(r, rcarry):
            wv = plsc.load_gather(w_sl, [jnp.full((16,), l * G + r, jnp.int32)])
            for k in range(CW // 16):
                sbuf[r, pl.ds(k * 16, 16)] = wv
            return rcarry
        lax.fori_loop(0, G, _r_body, 0)

    def _gather(c, l, buf, sem):
        pltpu.async_copy(zs[c].at[src_v.at[pl.ds(l * G, G)]], buf, sem)

    def _wait_gather(c, l, buf, sem):
        pltpu.make_async_copy(zs[c].at[src_v.at[pl.ds(l * G, G)]],
                              buf, sem).wait()

    def _stage(l, m):
        pltpu.async_copy(dst_hbm.at[gstart + l], db_v.at[m], sds[m])

    def _wait_stage(l, m):
        pltpu.make_async_copy(dst_hbm.at[gstart + l], db_v.at[m],
                              sds[m]).wait()

    def _scatter(buf, m, sem):
        pltpu.async_copy(buf, acc.at[db_v.at[m, 0]], sem, add=True)

    def _wait_scatter(buf, m, sem):
        pltpu.make_async_copy(buf, acc.at[db_v.at[m, 0]], sem).wait()

    for c in range(NCHUNK + 1):
        is_den = c == NCHUNK
        pltpu.sync_copy(zer_hbm.at[pl.ds(row0, RPT)],
                        acc.at[pl.ds(row0, RPT)])
        plsc.subcore_barrier()

        if not is_den:
            _gather(c, 0, g0_v, gs0)
            _gather(c, 1, g1_v, gs1)
        _stage(0, 0)
        _stage(1, 1)

        def _u_body(u, carry):
            for half in range(4):
                par = half % 2
                gbuf, gsem = (g0_v, gs0) if par == 0 else (g1_v, gs1)
                sbuf, ssem = (s0_v, ss0) if par == 0 else (s1_v, ss1)
                l = 4 * u + half

                @pl.when(l < k_here)
                def _(l=l, gbuf=gbuf, gsem=gsem, sbuf=sbuf, ssem=ssem,
                      slot=half):
                    if not is_den:
                        _wait_gather(c, l, gbuf, gsem)
                    _wait_stage(l, slot)

                    @pl.when(l >= 2)
                    def _():
                        _wait_scatter(sbuf, (slot + 2) % 4, ssem)

                    @pl.when(l + 2 < k_here)
                    def _():
                        _stage(l + 2, (slot + 2) % 4)
                    if is_den:
                        _fill(l, sbuf)
                    else:
                        _scale_into(l, gbuf, sbuf)
                    _scatter(sbuf, slot, ssem)

                    if not is_den:
                        @pl.when(l + 2 < k_here)
                        def _():
                            _gather(c, l + 2, gbuf, gsem)
            return carry
        lax.fori_loop(0, (KMAX + 3) // 4, _u_body, 0)

        _wait_scatter(s0_v, 0, ss0)
        _wait_scatter(s1_v, 1, ss1)

        plsc.subcore_barrier()
        if not is_den:
            pltpu.sync_copy(acc.at[pl.ds(row0, RPT)],
                            num_hbm.at[cid, c, pl.ds(row0, RPT)])
            plsc.subcore_barrier()
        else:
            pltpu.sync_copy(acc.at[pl.ds(row0, RPT)],
                            den_hbm.at[cid, pl.ds(row0, RPT)])


_sc_aggregate = functools.partial(
    pl.kernel,
    out_type=[jax.ShapeDtypeStruct((NC, NCHUNK, NPAD, CW), jnp.float32),
              jax.ShapeDtypeStruct((NC, NPAD, CW), jnp.float32)],
    mesh=plsc.VectorSubcoreMesh(core_axis_name="c", subcore_axis_name="s"),
    compiler_params=pltpu.CompilerParams(needs_layout_passes=False),
    scratch_types=[
        pltpu.VMEM((KMAX * G,), jnp.int32),
        pltpu.VMEM((KMAX * G,), jnp.float32),
        pltpu.VMEM((G, CW), jnp.float32),
        pltpu.VMEM((G, CW), jnp.float32),
        pltpu.VMEM((G, CW), jnp.float32),
        pltpu.VMEM((G, CW), jnp.float32),
        pltpu.VMEM((4, 1, G), jnp.int32),
        pltpu.VMEM_SHARED((NPAD, CW), jnp.float32),
        pltpu.SemaphoreType.DMA,
        pltpu.SemaphoreType.DMA,
        pltpu.SemaphoreType.DMA,
        pltpu.SemaphoreType.DMA,
        pltpu.SemaphoreType.DMA,
        pltpu.SemaphoreType.DMA,
        pltpu.SemaphoreType.DMA,
        pltpu.SemaphoreType.DMA,
    ],
)(_scb_body)


def _comb_body(num_ref, den_ref, out_ref):
    d = den_ref[0, :, 0:1] + den_ref[1, :, 0:1]
    good = d > 0
    dsafe = jnp.where(good, d, jnp.float32(1.0))
    for c in range(NCHUNK):
        n = num_ref[0, c] + num_ref[1, c]
        out_ref[:, c * CW:(c + 1) * CW] = jnp.where(good, n / dsafe,
                                                    jnp.float32(0.0))


def _tc_combine(num, den):
    return pl.pallas_call(
        _comb_body,
        grid=(GRID,),
        in_specs=[
            pl.BlockSpec((NC, NCHUNK, RB, CW), lambda i: (0, 0, i, 0)),
            pl.BlockSpec((NC, RB, CW), lambda i: (0, i, 0)),
        ],
        out_specs=pl.BlockSpec((RB, OUT_F), lambda i: (i, 0)),
        out_shape=jax.ShapeDtypeStruct((N, OUT_F), jnp.float32),
    )(num, den)


def kernel(h, edge_index, W, a):
    src = edge_index[0]
    dst = edge_index[1]
    pad = jnp.full((EPAD - E,), NPAD - 1, jnp.int32)
    src1 = jnp.concatenate([src, pad])
    dst1 = jnp.concatenate([dst, pad])
    src3 = src1.reshape(NW, NGA, GA)
    dst3 = dst1.reshape(NW, NGA, GA)
    dst2 = dst1.reshape(NGROUPS, 1, G)
    A2 = jnp.concatenate([a[:OUT_F], a[OUT_F:]], axis=1)

    z0, z1, z2, z3, s = _tc_matmul(h, W, A2)
    w2 = _sc_weights(s[0], s[1], src3, dst3)
    zer = jnp.zeros((NPAD, CW), jnp.float32)
    num, den = _sc_aggregate(z0, z1, z2, z3, w2, src1, dst2, zer)
    return _tc_combine(num, den)
